# Initial kernel scaffold; baseline (speedup 1.0000x reference)
import jax
import jax.numpy as jnp
from jax.experimental import pallas as pl


def kernel(x, edge_index, W_self1, W_neigh1, b1, W_self2, W_neigh2, b2):
    raise NotImplementedError("write your pallas kernel here")


# speedup vs baseline: 5.1561x; 5.1561x over previous
import jax
import jax.numpy as jnp
from jax import lax
from jax.experimental import pallas as pl
from jax.experimental.pallas import tpu as pltpu
from jax.experimental.pallas import tpu_sc as plsc

N = 10000
E = 320000
D = 128
H = 128
C = 64

NC, NS = 2, 16
NW = NC * NS
ET = E // NW
CH = 128
NCHUNK = 79
ETP = NCHUNK * CH
NP = 10112
RPT = NP // NS

_mesh = plsc.VectorSubcoreMesh(core_axis_name="c", subcore_axis_name="s")


def _sc_deg_body(ei_hbm, zdeg_hbm, ones_hbm, deg_hbm,
                 dst_v, ones_v, deg_sh):
    cid = lax.axis_index("c")
    sid = lax.axis_index("s")
    w = cid * NS + sid
    r0 = sid * RPT
    pltpu.sync_copy(ei_hbm.at[w, 1], dst_v)
    pltpu.sync_copy(ones_hbm, ones_v)
    pltpu.sync_copy(zdeg_hbm, deg_sh.at[pl.ds(r0, RPT)])
    plsc.subcore_barrier()

    def step(j, _):
        pltpu.sync_copy(ones_v, deg_sh.at[dst_v.at[j]], add=True)
        return 0

    lax.fori_loop(0, NCHUNK, step, 0)
    plsc.subcore_barrier()
    pltpu.sync_copy(deg_sh.at[pl.ds(r0, RPT)],
                    deg_hbm.at[cid, pl.ds(r0, RPT)])


def _sc_agg_body(z_hbm, ei_hbm, zrow_hbm, agg_hbm,
                 ei_v, rows_v, acc_sh, sem):
    cid = lax.axis_index("c")
    sid = lax.axis_index("s")
    w = cid * NS + sid
    r0 = sid * RPT

    pltpu.sync_copy(ei_hbm.at[w], ei_v)
    pltpu.sync_copy(zrow_hbm, acc_sh.at[pl.ds(r0, RPT)])
    plsc.subcore_barrier()

    def step(j, _):
        pltpu.async_copy(z_hbm.at[ei_v.at[0, j]], rows_v, sem).wait()
        pltpu.sync_copy(rows_v, acc_sh.at[ei_v.at[1, j]], add=True)
        return 0

    lax.fori_loop(0, NCHUNK, step, 0)
    plsc.subcore_barrier()

    pltpu.sync_copy(acc_sh.at[pl.ds(r0, RPT)], agg_hbm.at[cid, pl.ds(r0, RPT)])


def _sc_degree(ei4):
    fn = pl.kernel(
        _sc_deg_body,
        out_type=(jax.ShapeDtypeStruct((NC, NP, H), jnp.float32),),
        mesh=_mesh,
        scratch_types=(
            pltpu.VMEM((NCHUNK, CH), jnp.int32),
            pltpu.VMEM((CH, H), jnp.float32),
            pltpu.VMEM_SHARED((NP, H), jnp.float32),
        ),
    )
    return fn(ei4, jnp.zeros((RPT, H), jnp.float32),
              jnp.ones((CH, H), jnp.float32))[0]


def _sc_aggregate(z, ei4):
    fn = pl.kernel(
        _sc_agg_body,
        out_type=(jax.ShapeDtypeStruct((NC, NP, H), jnp.float32),),
        mesh=_mesh,
        scratch_types=(
            pltpu.VMEM((2, NCHUNK, CH), jnp.int32),
            pltpu.VMEM((CH, H), jnp.float32),
            pltpu.VMEM_SHARED((NP, H), jnp.float32),
            pltpu.SemaphoreType.DMA,
        ),
    )
    return fn(z, ei4, jnp.zeros((RPT, H), jnp.float32))[0]


_BM = 2000


def _tc_a_body(x_ref, w_ref, o_ref):
    o_ref[...] = jnp.dot(x_ref[...], w_ref[...],
                         preferred_element_type=jnp.float32,
                         precision=lax.Precision.HIGHEST)


def _tc_b_body(x_ref, agg_ref, deg_ref, ws1_ref, b1_ref, wn2_ref,
               hid_ref, z2_ref):
    deg = jnp.maximum(deg_ref[0, :, 0:1] + deg_ref[1, :, 0:1], 1.0)
    hn = (agg_ref[0] + agg_ref[1]) / deg
    hid = x_ref[...] @ ws1_ref[...] + hn + b1_ref[...]
    hid = jnp.maximum(hid, 0.0)
    hid_ref[...] = hid
    z2 = jnp.dot(hid, wn2_ref[...],
                 preferred_element_type=jnp.float32,
                 precision=lax.Precision.HIGHEST)
    z2_ref[...] = jnp.concatenate([z2, jnp.zeros_like(z2)], axis=1)


def _tc_c_body(hid_ref, agg_ref, deg_ref, ws2_ref, b2_ref, o_ref):
    deg = jnp.maximum(deg_ref[0, :, 0:1] + deg_ref[1, :, 0:1], 1.0)
    hn = (agg_ref[0, :, :C] + agg_ref[1, :, :C]) / deg
    o_ref[...] = hid_ref[...] @ ws2_ref[...] + hn + b2_ref[...]


def kernel(x, edge_index, W_self1, W_neigh1, b1, W_self2, W_neigh2, b2):
    ei = edge_index.astype(jnp.int32).reshape(2, NW, ET)
    pad_src = jnp.zeros((1, NW, ETP - ET), jnp.int32)
    pad_dst = jnp.full((1, NW, ETP - ET), NP - 1, jnp.int32)
    ei4 = jnp.concatenate([ei, jnp.concatenate([pad_src, pad_dst], 0)], 2)
    ei4 = jnp.swapaxes(ei4, 0, 1).reshape(NW, 2, NCHUNK, CH)
    b1r = b1.reshape(1, H)
    b2r = b2.reshape(1, C)

    deg = _sc_degree(ei4)

    z1 = pl.pallas_call(
        _tc_a_body,
        grid=(N // _BM,),
        in_specs=[pl.BlockSpec((_BM, D), lambda i: (i, 0)),
                  pl.BlockSpec((D, H), lambda i: (0, 0))],
        out_specs=pl.BlockSpec((_BM, H), lambda i: (i, 0)),
        out_shape=jax.ShapeDtypeStruct((N, H), jnp.float32),
    )(x, W_neigh1)

    agg1 = _sc_aggregate(z1, ei4)

    hidden, z2 = pl.pallas_call(
        _tc_b_body,
        grid=(N // _BM,),
        in_specs=[pl.BlockSpec((_BM, D), lambda i: (i, 0)),
                  pl.BlockSpec((NC, _BM, H), lambda i: (0, i, 0)),
                  pl.BlockSpec((NC, _BM, H), lambda i: (0, i, 0)),
                  pl.BlockSpec((D, H), lambda i: (0, 0)),
                  pl.BlockSpec((1, H), lambda i: (0, 0)),
                  pl.BlockSpec((H, C), lambda i: (0, 0))],
        out_specs=[pl.BlockSpec((_BM, H), lambda i: (i, 0)),
                   pl.BlockSpec((_BM, H), lambda i: (i, 0))],
        out_shape=[jax.ShapeDtypeStruct((N, H), jnp.float32),
                   jax.ShapeDtypeStruct((N, H), jnp.float32)],
    )(x, agg1, deg, W_self1, b1r, W_neigh2)

    agg2 = _sc_aggregate(z2, ei4)

    logits = pl.pallas_call(
        _tc_c_body,
        grid=(N // _BM,),
        in_specs=[pl.BlockSpec((_BM, H), lambda i: (i, 0)),
                  pl.BlockSpec((NC, _BM, H), lambda i: (0, i, 0)),
                  pl.BlockSpec((NC, _BM, H), lambda i: (0, i, 0)),
                  pl.BlockSpec((H, C), lambda i: (0, 0)),
                  pl.BlockSpec((1, C), lambda i: (0, 0))],
        out_specs=pl.BlockSpec((_BM, C), lambda i: (i, 0)),
        out_shape=jax.ShapeDtypeStruct((N, C), jnp.float32),
    )(hidden, agg2, deg, W_self2, b2r)
    return logits
